# Initial kernel scaffold; baseline (speedup 1.0000x reference)
import jax
import jax.numpy as jnp
from jax.experimental import pallas as pl


def kernel(x, edge_index, edge_type, basis1, comp1, w_self1, eps1, basis2, comp2, w_self2, eps2):
    raise NotImplementedError("write your pallas kernel here")


# speedup vs baseline: 6.1629x; 6.1629x over previous
import functools

import jax
import jax.numpy as jnp
from jax import lax
from jax.experimental import pallas as pl
from jax.experimental.pallas import tpu as pltpu
from jax.experimental.pallas import tpu_sc as plsc

N = 10000
E = 320000
IN_DIM = 128
HID_DIM = 256
OUT_DIM = 64
NUM_REL = 8
NUM_BASES = 4

NCORES = 2
NTILES = 16
CHUNK = 128
NROWS = 10112
ROWS_PER_TILE = NROWS // NTILES
PAD_DST = 10100
E_PAD = 323584


def _make_sc_agg(d: int, n_table_rows: int, row_stride: int, edge_split: bool):
  per_tile = E_PAD // (NTILES * NCORES) if edge_split else E_PAD // NTILES
  n_chunks = per_tile // CHUNK
  mesh = plsc.VectorSubcoreMesh(core_axis_name="c", subcore_axis_name="s")

  @functools.partial(
      pl.kernel,
      out_type=jax.ShapeDtypeStruct((NCORES * NROWS, d), jnp.float32),
      mesh=mesh,
      scratch_types=[
          pltpu.VMEM((CHUNK,), jnp.int32),
          pltpu.VMEM((CHUNK,), jnp.int32),
          pltpu.VMEM((CHUNK,), jnp.int32),
          pltpu.VMEM((CHUNK,), jnp.int32),
          pltpu.VMEM((CHUNK, d), jnp.float32),
          pltpu.VMEM_SHARED((NROWS, d), jnp.float32),
          pltpu.SemaphoreType.DMA,
      ],
  )
  def sc_agg(table_h, src_h, typ_h, dst_h, zeros_h, out_h,
             srcv, typv, dstv, gidx, rows, accum, sem):
    cid = lax.axis_index("c")
    sid = lax.axis_index("s")
    r0 = sid * ROWS_PER_TILE
    pltpu.sync_copy(zeros_h.at[pl.ds(r0, ROWS_PER_TILE)],
                    accum.at[pl.ds(r0, ROWS_PER_TILE)])
    plsc.subcore_barrier()

    if edge_split:
      base = (sid * NCORES + cid) * per_tile
      idx_off = jnp.int32(0)
    else:
      base = sid * per_tile
      idx_off = cid * jnp.int32(NUM_REL * n_table_rows)

    def body(g, carry):
      off = base + g * CHUNK
      pltpu.sync_copy(src_h.at[pl.ds(off, CHUNK)], srcv)
      pltpu.sync_copy(typ_h.at[pl.ds(off, CHUNK)], typv)
      pltpu.sync_copy(dst_h.at[pl.ds(off, CHUNK)], dstv)
      for j in range(CHUNK // 16):
        sl = pl.ds(j * 16, 16)
        gidx[sl] = typv[sl] * jnp.int32(row_stride) + srcv[sl] + idx_off
      pltpu.async_copy(table_h.at[gidx], rows, sem).wait()
      pltpu.sync_copy(rows, accum.at[dstv], add=True)
      return carry

    lax.fori_loop(0, n_chunks, body, 0)
    plsc.subcore_barrier()
    pltpu.sync_copy(accum.at[pl.ds(r0, ROWS_PER_TILE)],
                    out_h.at[pl.ds(cid * NROWS + r0, ROWS_PER_TILE)])

  return sc_agg


_sc_agg_l1 = _make_sc_agg(IN_DIM, N, N, edge_split=False)
_sc_agg_l2 = _make_sc_agg(IN_DIM, NROWS, NROWS, edge_split=True)


NB1 = 1000
NB2 = 1264


def _l1_dense_body(x_ref, basis_ref, comp_ref, wself_ref, tbl_ref, self_ref):
  x = x_ref[...]
  xb = [jnp.dot(x, basis_ref[b], preferred_element_type=jnp.float32)
        for b in range(NUM_BASES)]
  for r in range(NUM_REL):
    tr = xb[0] * comp_ref[r, 0]
    for b in range(1, NUM_BASES):
      tr = tr + xb[b] * comp_ref[r, b]
    tbl_ref[0, r] = tr[:, :IN_DIM]
    tbl_ref[1, r] = tr[:, IN_DIM:]
  self_ref[...] = jnp.dot(x, wself_ref[...], preferred_element_type=jnp.float32)


def _l2_dense_body(agg_ref, self1_ref, basis_ref, comp_ref, wself_ref,
                   tbl_ref, self2_ref):
  h0 = jnp.maximum(agg_ref[0] + self1_ref[:, :IN_DIM], 0.0)
  h1 = jnp.maximum(agg_ref[1] + self1_ref[:, IN_DIM:], 0.0)
  h = jnp.concatenate([h0, h1], axis=1)
  hb = [jnp.dot(h, basis_ref[b], preferred_element_type=jnp.float32)
        for b in range(NUM_BASES)]
  for r in range(NUM_REL):
    tr = hb[0] * comp_ref[r, 0]
    for b in range(1, NUM_BASES):
      tr = tr + hb[b] * comp_ref[r, b]
    tbl_ref[r] = jnp.concatenate([tr, jnp.zeros_like(tr)], axis=1)
  self2_ref[...] = jnp.dot(h, wself_ref[...],
                           preferred_element_type=jnp.float32)


def _final_body(a0_ref, a1_ref, s_ref, o_ref):
  o_ref[...] = (a0_ref[...][:, :OUT_DIM] + a1_ref[...][:, :OUT_DIM]
                + s_ref[...])


def kernel(x, edge_index, edge_type, basis1, comp1, w_self1, eps1,
           basis2, comp2, w_self2, eps2):
  src = edge_index[0].astype(jnp.int32)
  dst = edge_index[1].astype(jnp.int32)
  typ = edge_type.astype(jnp.int32)
  npad = E_PAD - E
  src_p = jnp.concatenate([src, jnp.zeros((npad,), jnp.int32)])
  dst_p = jnp.concatenate([dst, jnp.full((npad,), PAD_DST, jnp.int32)])
  typ_p = jnp.concatenate([typ, jnp.zeros((npad,), jnp.int32)])
  zeros1 = jnp.zeros((NROWS, IN_DIM), jnp.float32)
  wself1_eff = (1.0 + eps1) * w_self1
  wself2_eff = (1.0 + eps2) * w_self2

  tbl1, self1 = pl.pallas_call(
      _l1_dense_body,
      grid=(N // NB1,),
      in_specs=[
          pl.BlockSpec((NB1, IN_DIM), lambda i: (i, 0)),
          pl.BlockSpec((NUM_BASES, IN_DIM, HID_DIM), lambda i: (0, 0, 0)),
          pl.BlockSpec(memory_space=pltpu.SMEM),
          pl.BlockSpec((IN_DIM, HID_DIM), lambda i: (0, 0)),
      ],
      out_specs=[
          pl.BlockSpec((2, NUM_REL, NB1, IN_DIM), lambda i: (0, 0, i, 0)),
          pl.BlockSpec((NB1, HID_DIM), lambda i: (i, 0)),
      ],
      out_shape=[
          jax.ShapeDtypeStruct((2, NUM_REL, N, IN_DIM), jnp.float32),
          jax.ShapeDtypeStruct((N, HID_DIM), jnp.float32),
      ],
  )(x, basis1, comp1, wself1_eff)

  agg1_flat = _sc_agg_l1(tbl1.reshape(2 * NUM_REL * N, IN_DIM),
                         src_p, typ_p, dst_p, zeros1)
  agg1 = agg1_flat.reshape(2, NROWS, IN_DIM)

  self1_pad = jnp.zeros((NROWS, HID_DIM), jnp.float32).at[:N].set(self1)
  tbl2, self2 = pl.pallas_call(
      _l2_dense_body,
      grid=(NROWS // NB2,),
      in_specs=[
          pl.BlockSpec((2, NB2, IN_DIM), lambda i: (0, i, 0)),
          pl.BlockSpec((NB2, HID_DIM), lambda i: (i, 0)),
          pl.BlockSpec((NUM_BASES, HID_DIM, OUT_DIM), lambda i: (0, 0, 0)),
          pl.BlockSpec(memory_space=pltpu.SMEM),
          pl.BlockSpec((HID_DIM, OUT_DIM), lambda i: (0, 0)),
      ],
      out_specs=[
          pl.BlockSpec((NUM_REL, NB2, IN_DIM), lambda i: (0, i, 0)),
          pl.BlockSpec((NB2, OUT_DIM), lambda i: (i, 0)),
      ],
      out_shape=[
          jax.ShapeDtypeStruct((NUM_REL, NROWS, IN_DIM), jnp.float32),
          jax.ShapeDtypeStruct((NROWS, OUT_DIM), jnp.float32),
      ],
  )(agg1, self1_pad, basis2, comp2, wself2_eff)

  agg2_flat = _sc_agg_l2(tbl2.reshape(NUM_REL * NROWS, IN_DIM),
                         src_p, typ_p, dst_p, zeros1)
  agg2 = agg2_flat.reshape(2, NROWS, IN_DIM)

  out_pad = pl.pallas_call(
      _final_body,
      grid=(NROWS // NB2,),
      in_specs=[
          pl.BlockSpec((NB2, IN_DIM), lambda i: (i, 0)),
          pl.BlockSpec((NB2, IN_DIM), lambda i: (i, 0)),
          pl.BlockSpec((NB2, OUT_DIM), lambda i: (i, 0)),
      ],
      out_specs=pl.BlockSpec((NB2, OUT_DIM), lambda i: (i, 0)),
      out_shape=jax.ShapeDtypeStruct((NROWS, OUT_DIM), jnp.float32),
  )(agg2[0], agg2[1], self2)
  return out_pad[:N]
